# initial kernel scaffold (unmeasured)
import jax
import jax.numpy as jnp
from jax import lax
from jax.experimental import pallas as pl
from jax.experimental.pallas import tpu as pltpu

N_Y = 4


def kernel(Q, K, V):
    b, q_len, h, d = Q.shape
    _, kv, _, _ = K.shape
    scale = d ** -0.5

    def partial_body(q_ref, k_ref, v_ref, l_ref, u_ref):
        q = q_ref[0, :, 0, :].astype(jnp.bfloat16)
        k = k_ref[0, :, 0, :].astype(jnp.bfloat16)
        v = v_ref[0, :, 0, :].astype(jnp.bfloat16)
        s = lax.dot_general(
            q, k, (((1,), (1,)), ((), ())),
            preferred_element_type=jnp.float32,
        ) * scale
        p = jnp.exp(s)
        l = jnp.sum(p, axis=1)
        u = lax.dot_general(
            p.astype(jnp.bfloat16), v, (((1,), (0,)), ((), ())),
            preferred_element_type=jnp.float32,
        )
        l_ref[0, :, 0] = l
        u_ref[0, :, 0, :] = u

    l_part, u_part = pl.pallas_call(
        partial_body,
        grid=(b, h),
        in_specs=[
            pl.BlockSpec((1, q_len, 1, d), lambda i, j: (i, 0, j, 0)),
            pl.BlockSpec((1, kv, 1, d), lambda i, j: (i, 0, j, 0)),
            pl.BlockSpec((1, kv, 1, d), lambda i, j: (i, 0, j, 0)),
        ],
        out_specs=[
            pl.BlockSpec((1, q_len, 1), lambda i, j: (i, 0, j)),
            pl.BlockSpec((1, q_len, 1, d), lambda i, j: (i, 0, j, 0)),
        ],
        out_shape=[
            jax.ShapeDtypeStruct((b, q_len, h), jnp.float32),
            jax.ShapeDtypeStruct((b, q_len, h, d), jnp.float32),
        ],
    )(Q, K, V)

    def allreduce_body(l_ref, u_ref, o_ref, lc, uc, l_acc, u_acc,
                       send_sems, recv_sems):
        my_x = lax.axis_index("x")
        my_y = lax.axis_index("y")
        my_z = lax.axis_index("z")
        peer0 = (my_x, my_y ^ 1, my_z)
        peer1 = (my_x, my_y ^ 2, my_z)

        barrier = pltpu.get_barrier_semaphore()
        for peer in (peer0, peer1):
            pl.semaphore_signal(
                barrier, inc=1, device_id=peer,
                device_id_type=pl.DeviceIdType.MESH,
            )
        pl.semaphore_wait(barrier, 2)

        rdl0 = pltpu.make_async_remote_copy(
            src_ref=l_ref, dst_ref=lc.at[0],
            send_sem=send_sems.at[0], recv_sem=recv_sems.at[0],
            device_id=peer0, device_id_type=pl.DeviceIdType.MESH,
        )
        rdu0 = pltpu.make_async_remote_copy(
            src_ref=u_ref, dst_ref=uc.at[0],
            send_sem=send_sems.at[1], recv_sem=recv_sems.at[1],
            device_id=peer0, device_id_type=pl.DeviceIdType.MESH,
        )
        rdl0.start()
        rdu0.start()
        rdl0.wait()
        rdu0.wait()
        l_acc[...] = l_ref[...] + lc[0]
        u_acc[...] = u_ref[...] + uc[0]

        rdl1 = pltpu.make_async_remote_copy(
            src_ref=l_acc, dst_ref=lc.at[1],
            send_sem=send_sems.at[2], recv_sem=recv_sems.at[2],
            device_id=peer1, device_id_type=pl.DeviceIdType.MESH,
        )
        rdu1 = pltpu.make_async_remote_copy(
            src_ref=u_acc, dst_ref=uc.at[1],
            send_sem=send_sems.at[3], recv_sem=recv_sems.at[3],
            device_id=peer1, device_id_type=pl.DeviceIdType.MESH,
        )
        rdl1.start()
        rdu1.start()
        rdl1.wait()
        rdu1.wait()

        total_l = l_acc[...] + lc[1]
        total_u = u_acc[...] + uc[1]
        o_ref[...] = total_u / total_l[..., None]

    return pl.pallas_call(
        allreduce_body,
        out_shape=jax.ShapeDtypeStruct((b, q_len, h, d), jnp.float32),
        in_specs=[
            pl.BlockSpec(memory_space=pltpu.VMEM),
            pl.BlockSpec(memory_space=pltpu.VMEM),
        ],
        out_specs=pl.BlockSpec(memory_space=pltpu.VMEM),
        scratch_shapes=[
            pltpu.VMEM((2, b, q_len, h), jnp.float32),
            pltpu.VMEM((2, b, q_len, h, d), jnp.float32),
            pltpu.VMEM((b, q_len, h), jnp.float32),
            pltpu.VMEM((b, q_len, h, d), jnp.float32),
            pltpu.SemaphoreType.DMA((4,)),
            pltpu.SemaphoreType.DMA((4,)),
        ],
        compiler_params=pltpu.CompilerParams(collective_id=0),
    )(l_part, u_part)


# baseline (device time: 188715 ns/iter reference)
import jax
import jax.numpy as jnp
from jax import lax
from jax.experimental import pallas as pl
from jax.experimental.pallas import tpu as pltpu

KV_CHUNK = 1024


def kernel(Q, K, V):
    b, q_len, h, d = Q.shape
    _, kv, _, _ = K.shape
    scale = d ** -0.5
    n_chunks = kv // KV_CHUNK

    def partial_body(q_ref, k_ref, v_ref, ul_ref, acc):
        c = pl.program_id(1)

        @pl.when(c == 0)
        def _():
            acc[...] = jnp.zeros_like(acc)

        for head in range(h):
            q = q_ref[0, :, head, :].astype(jnp.bfloat16)
            k = k_ref[0, :, head, :].astype(jnp.bfloat16)
            v = v_ref[0, :, head, :].astype(jnp.bfloat16)
            s = lax.dot_general(
                q, k, (((1,), (1,)), ((), ())),
                preferred_element_type=jnp.float32,
            ) * scale
            p = jnp.exp(s)
            u = lax.dot_general(
                p.astype(jnp.bfloat16), v, (((1,), (0,)), ((), ())),
                preferred_element_type=jnp.float32,
            )
            l = jnp.sum(p, axis=1)
            acc[:, head, :d] += u
            acc[:, head, d:] += jnp.broadcast_to(l[:, None], (q_len, d))

        @pl.when(c == n_chunks - 1)
        def _():
            ul_ref[0] = acc[...]

    ul_part = pl.pallas_call(
        partial_body,
        grid=(b, n_chunks),
        in_specs=[
            pl.BlockSpec((1, q_len, h, d), lambda i, c: (i, 0, 0, 0)),
            pl.BlockSpec((1, KV_CHUNK, h, d), lambda i, c: (i, c, 0, 0)),
            pl.BlockSpec((1, KV_CHUNK, h, d), lambda i, c: (i, c, 0, 0)),
        ],
        out_specs=pl.BlockSpec((1, q_len, h, 2 * d), lambda i, c: (i, 0, 0, 0)),
        out_shape=jax.ShapeDtypeStruct((b, q_len, h, 2 * d), jnp.float32),
        scratch_shapes=[
            pltpu.VMEM((q_len, h, 2 * d), jnp.float32),
        ],
    )(Q, K, V)

    def allreduce_body(ul_ref, o_ref, comm, acc, send_sems, recv_sems):
        my_x = lax.axis_index("x")
        my_y = lax.axis_index("y")
        my_z = lax.axis_index("z")
        peer0 = (my_x, my_y ^ 1, my_z)
        peer1 = (my_x, my_y ^ 2, my_z)

        barrier = pltpu.get_barrier_semaphore()
        for peer in (peer0, peer1):
            pl.semaphore_signal(
                barrier, inc=1, device_id=peer,
                device_id_type=pl.DeviceIdType.MESH,
            )
        pl.semaphore_wait(barrier, 2)

        rdma0 = pltpu.make_async_remote_copy(
            src_ref=ul_ref, dst_ref=comm.at[0],
            send_sem=send_sems.at[0], recv_sem=recv_sems.at[0],
            device_id=peer0, device_id_type=pl.DeviceIdType.MESH,
        )
        rdma0.start()
        rdma0.wait()
        acc[...] = ul_ref[...] + comm[0]

        rdma1 = pltpu.make_async_remote_copy(
            src_ref=acc, dst_ref=comm.at[1],
            send_sem=send_sems.at[1], recv_sem=recv_sems.at[1],
            device_id=peer1, device_id_type=pl.DeviceIdType.MESH,
        )
        rdma1.start()
        rdma1.wait()

        total = acc[...] + comm[1]
        d_ = o_ref.shape[-1]
        o_ref[...] = total[..., :d_] / total[..., d_:d_ + 1]

    return pl.pallas_call(
        allreduce_body,
        out_shape=jax.ShapeDtypeStruct((b, q_len, h, d), jnp.float32),
        in_specs=[pl.BlockSpec(memory_space=pltpu.VMEM)],
        out_specs=pl.BlockSpec(memory_space=pltpu.VMEM),
        scratch_shapes=[
            pltpu.VMEM((2, b, q_len, h, 2 * d), jnp.float32),
            pltpu.VMEM((b, q_len, h, 2 * d), jnp.float32),
            pltpu.SemaphoreType.DMA((2,)),
            pltpu.SemaphoreType.DMA((2,)),
        ],
        compiler_params=pltpu.CompilerParams(collective_id=0),
    )(ul_part)


# device time: 63027 ns/iter; 2.9942x vs baseline; 2.9942x over previous
import jax
import jax.numpy as jnp
from jax import lax
from jax.experimental import pallas as pl
from jax.experimental.pallas import tpu as pltpu

N_SPLIT = 8


def kernel(Q, K, V):
    b, q_len, h, d = Q.shape
    _, kv, _, _ = K.shape
    scale = d ** -0.5
    kv_slice = kv // N_SPLIT

    my_x = lax.axis_index("x")
    my_y = lax.axis_index("y")
    my_z = lax.axis_index("z")
    xz = jnp.reshape(my_x * 4 + my_z, (1,)).astype(jnp.int32)

    def partial_body(xz_ref, q_ref, k_ref, v_ref, u_ref, l_ref):
        for head in range(h):
            q = q_ref[0, :, head, :].astype(jnp.bfloat16)
            k = k_ref[0, :, head, :].astype(jnp.bfloat16)
            v = v_ref[0, :, head, :].astype(jnp.bfloat16)
            s = lax.dot_general(
                q, k, (((1,), (1,)), ((), ())),
                preferred_element_type=jnp.float32,
            ) * scale
            p = jnp.exp(s)
            u = lax.dot_general(
                p.astype(jnp.bfloat16), v, (((1,), (0,)), ((), ())),
                preferred_element_type=jnp.float32,
            )
            u_ref[0, :, head, :] = u.astype(jnp.bfloat16)
            l_ref[0, :, head] = jnp.sum(p, axis=1)

    u_part, l_part = pl.pallas_call(
        partial_body,
        grid_spec=pltpu.PrefetchScalarGridSpec(
            num_scalar_prefetch=1,
            grid=(b,),
            in_specs=[
                pl.BlockSpec((1, q_len, h, d), lambda i, xz_r: (i, 0, 0, 0)),
                pl.BlockSpec((1, kv_slice, h, d),
                             lambda i, xz_r: (i, xz_r[0], 0, 0)),
                pl.BlockSpec((1, kv_slice, h, d),
                             lambda i, xz_r: (i, xz_r[0], 0, 0)),
            ],
            out_specs=[
                pl.BlockSpec((1, q_len, h, d), lambda i, xz_r: (i, 0, 0, 0)),
                pl.BlockSpec((1, q_len, h), lambda i, xz_r: (i, 0, 0)),
            ],
        ),
        out_shape=[
            jax.ShapeDtypeStruct((b, q_len, h, d), jnp.bfloat16),
            jax.ShapeDtypeStruct((b, q_len, h), jnp.float32),
        ],
    )(xz, Q, K, V)

    def allreduce_body(u_ref, l_ref, o_ref, u_send, l_send, u_comm, l_comm,
                       send_sems, recv_sems):
        mx = lax.axis_index("x")
        my = lax.axis_index("y")
        mz = lax.axis_index("z")
        peers = [
            (mx ^ 1, my, mz),
            (mx, my ^ 1, mz),
            (mx, my ^ 2, mz),
            (mx, my, mz ^ 1),
            (mx, my, mz ^ 2),
        ]

        barrier = pltpu.get_barrier_semaphore()
        for peer in peers:
            pl.semaphore_signal(
                barrier, inc=1, device_id=peer,
                device_id_type=pl.DeviceIdType.MESH,
            )
        pl.semaphore_wait(barrier, len(peers))

        u_acc = u_ref[...].astype(jnp.float32)
        l_acc = l_ref[...]
        for s, peer in enumerate(peers):
            if s == 0:
                src_u, src_l = u_ref, l_ref
            else:
                u_send[...] = u_acc.astype(jnp.bfloat16)
                l_send[...] = l_acc
                src_u, src_l = u_send, l_send
            rd_u = pltpu.make_async_remote_copy(
                src_ref=src_u, dst_ref=u_comm.at[s],
                send_sem=send_sems.at[2 * s], recv_sem=recv_sems.at[2 * s],
                device_id=peer, device_id_type=pl.DeviceIdType.MESH,
            )
            rd_l = pltpu.make_async_remote_copy(
                src_ref=src_l, dst_ref=l_comm.at[s],
                send_sem=send_sems.at[2 * s + 1],
                recv_sem=recv_sems.at[2 * s + 1],
                device_id=peer, device_id_type=pl.DeviceIdType.MESH,
            )
            rd_u.start()
            rd_l.start()
            rd_u.wait()
            rd_l.wait()
            u_acc = u_acc + u_comm[s].astype(jnp.float32)
            l_acc = l_acc + l_comm[s]

        o_ref[...] = u_acc / l_acc[..., None]

    return pl.pallas_call(
        allreduce_body,
        out_shape=jax.ShapeDtypeStruct((b, q_len, h, d), jnp.float32),
        in_specs=[
            pl.BlockSpec(memory_space=pltpu.VMEM),
            pl.BlockSpec(memory_space=pltpu.VMEM),
        ],
        out_specs=pl.BlockSpec(memory_space=pltpu.VMEM),
        scratch_shapes=[
            pltpu.VMEM((b, q_len, h, d), jnp.bfloat16),
            pltpu.VMEM((b, q_len, h), jnp.float32),
            pltpu.VMEM((5, b, q_len, h, d), jnp.bfloat16),
            pltpu.VMEM((5, b, q_len, h), jnp.float32),
            pltpu.SemaphoreType.DMA((10,)),
            pltpu.SemaphoreType.DMA((10,)),
        ],
        compiler_params=pltpu.CompilerParams(collective_id=0),
    )(u_part, l_part)
